# baseline (speedup 1.0000x reference)
import jax
import jax.numpy as jnp
from jax.experimental import pallas as pl
from jax.experimental.pallas import tpu as pltpu


def _copy_kernel(emb_ref, emb_out_ref):
    emb_out_ref[0, 0] = emb_ref[0, 0]


@jax.jit
def _run(emb_sentences, mask, W, b3):
    B, L, S, D = emb_sentences.shape
    C = W.shape[-1]
    BS = 2048
    grid = (B, L, S // BS)

    emb_out = pl.pallas_call(
        _copy_kernel,
        grid=grid,
        in_specs=[
            pl.BlockSpec((1, 1, BS, D), lambda bi, li, si: (bi, li, si, 0)),
        ],
        out_specs=pl.BlockSpec((1, 1, BS, D), lambda bi, li, si: (bi, li, si, 0)),
        out_shape=jax.ShapeDtypeStruct((B, L, S, D), jnp.float32),
        compiler_params=pltpu.CompilerParams(
            dimension_semantics=("parallel", "parallel", "parallel"),
        ),
    )(emb_sentences)
    logits = jnp.einsum("blsd,ldc->blsc", emb_sentences, W) + b3[:, None, :].reshape(1, L, 1, C)
    logits = logits + mask[:, None, :, :]
    return emb_out, logits


def kernel(emb_sentences, att_sentences, W, b):
    B, L, S, D = emb_sentences.shape
    mask = jnp.where(att_sentences, 0.0, -jnp.inf).astype(jnp.float32)
    mask = mask.reshape(B, S, 1)
    b3 = b.reshape(b.shape[0], 1, b.shape[1])
    emb_out, logits = _run(emb_sentences, mask, W, b3)
    return emb_out, att_sentences, logits


# speedup vs baseline: 1.1434x; 1.1434x over previous
import jax
import jax.numpy as jnp
from jax.experimental import pallas as pl
from jax.experimental.pallas import tpu as pltpu

_NBUF = 6
_K = 3
_NLG = 2


def _stream_kernel(mask_ref, w_ref, b_ref, emb_ref, emb_out_ref, logits_ref,
                   buf, lgbuf, sem_in, sem_out, sem_lg):
    n_tiles, S, D = emb_ref.shape
    L = w_ref.shape[0]

    def in_copy(i, slot):
        return pltpu.make_async_copy(emb_ref.at[i], buf.at[slot], sem_in.at[slot])

    def out_copy(i, slot):
        return pltpu.make_async_copy(buf.at[slot], emb_out_ref.at[i], sem_out.at[slot])

    def lg_copy(i, slot):
        return pltpu.make_async_copy(lgbuf.at[slot], logits_ref.at[i], sem_lg.at[slot])

    for j in range(_K):
        in_copy(j, j).start()

    def body(i, _):
        slot = jax.lax.rem(i, _NBUF)
        in_copy(i, slot).wait()
        out_copy(i, slot).start()

        @pl.when(i + _K < n_tiles)
        def _():
            nxt = i + _K
            slot2 = jax.lax.rem(nxt, _NBUF)

            @pl.when(nxt >= _NBUF)
            def _():
                out_copy(nxt - _NBUF, slot2).wait()

            in_copy(nxt, slot2).start()

        lyr = jax.lax.rem(i, L)
        bidx = jax.lax.div(i, L)
        lslot = jax.lax.rem(i, _NLG)

        @pl.when(i >= _NLG)
        def _():
            lg_copy(i - _NLG, lslot).wait()

        y = jnp.dot(buf[slot], w_ref[lyr], preferred_element_type=jnp.float32)
        lgbuf[lslot] = y + b_ref[lyr] + mask_ref[bidx]
        lg_copy(i, lslot).start()
        return 0

    jax.lax.fori_loop(0, n_tiles, body, 0)

    for c in range(max(0, n_tiles - _NBUF), n_tiles):
        out_copy(c, c % _NBUF).wait()
    for c in range(max(0, n_tiles - _NLG), n_tiles):
        lg_copy(c, c % _NLG).wait()


@jax.jit
def _run(emb_flat, mask, W, b3):
    T, S, D = emb_flat.shape
    L, _, C = W.shape

    emb_out, logits = pl.pallas_call(
        _stream_kernel,
        in_specs=[
            pl.BlockSpec(memory_space=pltpu.MemorySpace.VMEM),
            pl.BlockSpec(memory_space=pltpu.MemorySpace.VMEM),
            pl.BlockSpec(memory_space=pltpu.MemorySpace.VMEM),
            pl.BlockSpec(memory_space=pltpu.MemorySpace.HBM),
        ],
        out_specs=[
            pl.BlockSpec(memory_space=pltpu.MemorySpace.HBM),
            pl.BlockSpec(memory_space=pltpu.MemorySpace.HBM),
        ],
        out_shape=[
            jax.ShapeDtypeStruct((T, S, D), jnp.float32),
            jax.ShapeDtypeStruct((T, S, C), jnp.float32),
        ],
        scratch_shapes=[
            pltpu.VMEM((_NBUF, S, D), jnp.float32),
            pltpu.VMEM((_NLG, S, C), jnp.float32),
            pltpu.SemaphoreType.DMA((_NBUF,)),
            pltpu.SemaphoreType.DMA((_NBUF,)),
            pltpu.SemaphoreType.DMA((_NLG,)),
        ],
    )(mask, W, b3, emb_flat)
    return emb_out, logits


def kernel(emb_sentences, att_sentences, W, b):
    B, L, S, D = emb_sentences.shape
    C = W.shape[-1]
    mask = jnp.where(att_sentences, 0.0, -jnp.inf).astype(jnp.float32)
    mask = mask.reshape(B, S, 1)
    b3 = b.reshape(L, 1, C)
    emb_flat = emb_sentences.reshape(B * L, S, D)
    emb_out, logits = _run(emb_flat, mask, W, b3)
    return (emb_out.reshape(B, L, S, D), att_sentences,
            logits.reshape(B, L, S, C))
